# baseline (speedup 1.0000x reference)
import functools

import jax
import jax.numpy as jnp
from jax import lax
from jax.experimental import pallas as pl
from jax.experimental.pallas import tpu as pltpu
from jax.experimental.pallas import tpu_sc as plsc

NE, NN, NG = 160000, 10000, 16
NE_PAD = 163840
NN_PAD = 10240
DUMP = NN
NWORK = 32
G_EDGE = NE_PAD // 128
CE = 2048
CN = 2048
F32 = jnp.float32


def _gather_pallas(xr, xs, colg, rowg):
    W = xr.shape[1]
    GPT = G_EDGE // NWORK
    mesh = plsc.VectorSubcoreMesh(core_axis_name="c", subcore_axis_name="s")

    NSLOT = 2
    untiled = W < 128

    def body(xr_h, xs_h, col_h, row_h, gsum_h, col_v, row_v, *bufsem):
        bufa = bufsem[0:NSLOT]
        bufb = bufsem[NSLOT:2 * NSLOT]
        sems = bufsem[2 * NSLOT:3 * NSLOT]
        wid = lax.axis_index("s") * 2 + lax.axis_index("c")
        gbase = wid * GPT
        pltpu.sync_copy(col_h.at[pl.ds(gbase, GPT)], col_v)
        pltpu.sync_copy(row_h.at[pl.ds(gbase, GPT)], row_v)

        def fire(j, s):
            pltpu.async_copy(xr_h.at[col_v.at[j]], bufa[s], sems[s])
            pltpu.async_copy(xs_h.at[row_v.at[j]], bufb[s], sems[s])

        def drain_store(j, s):
            eb = (gbase + j) * 128
            pltpu.make_async_copy(xr_h.at[col_v.at[j]], bufa[s], sems[s]).wait()
            pltpu.make_async_copy(xs_h.at[row_v.at[j]], bufb[s], sems[s]).wait()

            def addrow(r, c):
                for q in range(W // 16):
                    sl = pl.ds(q * 16, 16)
                    bufa[s][r, sl] = bufa[s][r, sl] + bufb[s][r, sl]
                return c

            lax.fori_loop(0, 128, addrow, 0)
            pltpu.sync_copy(bufa[s], gsum_h.at[pl.ds(eb, 128)])

        for p in range(NSLOT - 1):
            fire(p, p)

        def step(j, c):
            for s in range(NSLOT):
                @pl.when(jnp.logical_and(j + NSLOT - 1 < GPT,
                                         (j + NSLOT - 1) % NSLOT == s))
                def _(s=s):
                    fire(j + NSLOT - 1, s)

            for s in range(NSLOT):
                @pl.when(j % NSLOT == s)
                def _(s=s):
                    drain_store(j, s)

            return c

        lax.fori_loop(0, GPT, step, 0)

    f = pl.kernel(
        body,
        out_type=jax.ShapeDtypeStruct((NE_PAD, W), F32),
        mesh=mesh,
        scratch_types=(
            [pltpu.VMEM((GPT, 128), jnp.int32)] * 2
            + [pltpu.VMEM((128, W), F32)] * (2 * NSLOT)
            + [pltpu.SemaphoreType.DMA] * NSLOT
        ),
        compiler_params=pltpu.CompilerParams(use_tc_tiling_on_sc=False)
        if untiled else None,
    )
    return f(xr, xs, colg, rowg)


def _scatter_pallas(e_new, colg, rowg, zeros):
    W = e_new.shape[1]
    RPT = NN_PAD // 16
    GPT = G_EDGE // 16
    untiled = W < 128
    mesh = plsc.VectorSubcoreMesh(core_axis_name="c", subcore_axis_name="s")

    def body(e_h, col_h, row_h, z_h, recv_h, send_h, acc, idx_v,
             ebuf0, ebuf1, seml0, seml1):
        cid = lax.axis_index("c")
        sid = lax.axis_index("s")
        pltpu.sync_copy(z_h.at[pl.ds(sid * RPT, RPT)], acc.at[pl.ds(sid * RPT, RPT)])

        @pl.when(cid == 0)
        def _():
            pltpu.sync_copy(col_h.at[pl.ds(sid * GPT, GPT)], idx_v)

        @pl.when(cid == 1)
        def _():
            pltpu.sync_copy(row_h.at[pl.ds(sid * GPT, GPT)], idx_v)

        plsc.subcore_barrier()

        def load(j, buf, sem):
            eb = (sid * GPT + j) * 128
            pltpu.async_copy(e_h.at[pl.ds(eb, 128)], buf, sem)

        def drain_scatter(j, buf, sem):
            eb = (sid * GPT + j) * 128
            pltpu.make_async_copy(e_h.at[pl.ds(eb, 128)], buf, sem).wait()
            pltpu.sync_copy(buf, acc.at[idx_v.at[j]], add=True)

        load(0, ebuf0, seml0)

        def step(j, c):
            @pl.when(jnp.logical_and(j + 1 < GPT, j % 2 == 0))
            def _():
                load(j + 1, ebuf1, seml1)

            @pl.when(jnp.logical_and(j + 1 < GPT, j % 2 == 1))
            def _():
                load(j + 1, ebuf0, seml0)

            @pl.when(j % 2 == 0)
            def _():
                drain_scatter(j, ebuf0, seml0)

            @pl.when(j % 2 == 1)
            def _():
                drain_scatter(j, ebuf1, seml1)

            return c

        lax.fori_loop(0, GPT, step, 0)
        plsc.subcore_barrier()

        @pl.when(cid == 0)
        def _():
            pltpu.sync_copy(acc.at[pl.ds(sid * RPT, RPT)], recv_h.at[pl.ds(sid * RPT, RPT)])

        @pl.when(cid == 1)
        def _():
            pltpu.sync_copy(acc.at[pl.ds(sid * RPT, RPT)], send_h.at[pl.ds(sid * RPT, RPT)])

    f = pl.kernel(
        body,
        out_type=(jax.ShapeDtypeStruct((NN_PAD, W), F32),
                  jax.ShapeDtypeStruct((NN_PAD, W), F32)),
        mesh=mesh,
        scratch_types=[
            pltpu.VMEM_SHARED((NN_PAD, W), F32),
            pltpu.VMEM((GPT, 128), jnp.int32),
            pltpu.VMEM((128, W), F32),
            pltpu.VMEM((128, W), F32),
            pltpu.SemaphoreType.DMA,
            pltpu.SemaphoreType.DMA,
        ],
        compiler_params=pltpu.CompilerParams(use_tc_tiling_on_sc=False)
        if untiled else None,
    )
    return f(e_new, colg, rowg, zeros)


def _layernorm(h, g, b):
    m = jnp.mean(h, axis=-1, keepdims=True)
    v = jnp.mean((h - m) ** 2, axis=-1, keepdims=True)
    return (h - m) * lax.rsqrt(v + 1e-5) * g + b


def _dot(a, b):
    return jnp.dot(a, b, preferred_element_type=F32)


def _dotp(a, b):
    return jnp.dot(a, b, preferred_element_type=F32,
                   precision=lax.Precision.HIGHEST)


def _proj_call(n_parts, Wr_parts, Ws_parts):
    lat = Wr_parts[0].shape[1]
    k = len(n_parts)
    grid = NN_PAD // CN

    def body(*refs):
        ns = refs[:k]
        wr = refs[k:2 * k]
        ws = refs[2 * k:3 * k]
        xr_r, xs_r = refs[3 * k], refs[3 * k + 1]
        xr = _dot(ns[0][...], wr[0][...])
        xs = _dot(ns[0][...], ws[0][...])
        for i in range(1, k):
            xr += _dot(ns[i][...], wr[i][...])
            xs += _dot(ns[i][...], ws[i][...])
        xr_r[...] = xr
        xs_r[...] = xs

    in_specs = (
        [pl.BlockSpec((CN, p.shape[1]), lambda i: (i, 0)) for p in n_parts]
        + [pl.BlockSpec(w.shape, lambda i: (0, 0)) for w in Wr_parts]
        + [pl.BlockSpec(w.shape, lambda i: (0, 0)) for w in Ws_parts]
    )
    out_specs = (pl.BlockSpec((CN, lat), lambda i: (i, 0)),) * 2
    return pl.pallas_call(
        body,
        grid=(grid,),
        in_specs=in_specs,
        out_specs=out_specs,
        out_shape=(jax.ShapeDtypeStruct((NN_PAD, lat), F32),) * 2,
    )(*n_parts, *Wr_parts, *Ws_parts)


def _edge_call(gsum, ea_parts, ohbr, u_cat, Wg, We_parts, b1, l2, ln):
    k = len(ea_parts)
    two_layer = l2 is not None
    dout = l2["W"].shape[1] if two_layer else b1.shape[1]
    grid = NE_PAD // CE

    def body(*refs):
        gsum_r = refs[0]
        eas = refs[1:1 + k]
        oh_r, u_r, wg_r = refs[1 + k], refs[2 + k], refs[3 + k]
        wes = refs[4 + k:4 + 2 * k]
        b1_r = refs[4 + 2 * k]
        pos = 5 + 2 * k
        if two_layer:
            w2_r, b2_r, g_r, be_r = refs[pos:pos + 4]
            pos += 4
        enew_r, e2g_r = refs[pos], refs[pos + 1]

        i = pl.program_id(0)
        h = gsum_r[...] + b1_r[...]
        for ear, wer in zip(eas, wes):
            h += _dot(ear[...], wer[...])
        ug = _dot(u_r[...], wg_r[...])
        oh = oh_r[...]
        h += _dotp(oh, ug)
        if two_layer:
            h = jnp.maximum(h, 0.0)
            h = _dot(h, w2_r[...]) + b2_r[...]
            h = jnp.maximum(h, 0.0)
            h = _layernorm(h, g_r[...], be_r[...])
        enew_r[...] = h
        contrib = lax.dot_general(oh, h, (((0,), (0,)), ((), ())),
                                  preferred_element_type=F32,
                                  precision=lax.Precision.HIGHEST)

        @pl.when(i == 0)
        def _():
            e2g_r[...] = contrib

        @pl.when(i > 0)
        def _():
            e2g_r[...] += contrib

    ins = [gsum, *ea_parts, ohbr, u_cat, Wg, *We_parts, b1]
    if two_layer:
        ins += [l2["W"], l2["b"], ln["g"], ln["b"]]
    in_specs = (
        [pl.BlockSpec((CE, gsum.shape[1]), lambda i: (i, 0))]
        + [pl.BlockSpec((CE, p.shape[1]), lambda i: (i, 0)) for p in ea_parts]
        + [pl.BlockSpec((CE, NG), lambda i: (i, 0))]
        + [pl.BlockSpec(a.shape, lambda i: (0, 0))
           for a in ins[2 + k:]]
    )
    out_specs = (pl.BlockSpec((CE, dout), lambda i: (i, 0)),
                 pl.BlockSpec((NG, dout), lambda i: (0, 0)))
    return pl.pallas_call(
        body,
        grid=(grid,),
        in_specs=in_specs,
        out_specs=out_specs,
        out_shape=(jax.ShapeDtypeStruct((NE_PAD, dout), F32),
                   jax.ShapeDtypeStruct((NG, dout), F32)),
    )(*ins)


def _pad_cols(w, to):
    return w if w.shape[1] == to else jnp.pad(w, ((0, 0), (0, to - w.shape[1])))


def _pad_rows(w, to):
    return w if w.shape[0] == to else jnp.pad(w, ((0, to - w.shape[0]), (0, 0)))


def _node_global_call(n_parts, ohb, recv, send, u_cat, e2g, np_, gp, ew,
                      projs=()):
    k = len(n_parts)
    nW1, nb1 = np_["layers"][0]["W"], np_["layers"][0]["b"]
    two_layer = np_["ln"] is not None or len(np_["layers"]) > 1
    nn = sum(p.shape[1] for p in n_parts)
    ng = u_cat.shape[1]
    ewd = recv.shape[1]
    Wn_n = [nW1[sum(p.shape[1] for p in n_parts[:i]):
                sum(p.shape[1] for p in n_parts[:i + 1])] for i in range(k)]
    Wn_g = nW1[nn:nn + ng]
    Wn_r = _pad_rows(nW1[nn + ng:nn + ng + ew], ewd)
    Wn_s = _pad_rows(nW1[nn + ng + ew:], ewd)
    nout = (np_["layers"][1]["W"] if two_layer else nW1).shape[1]

    gW1, gb1 = gp["layers"][0]["W"], gp["layers"][0]["b"]
    Wg_n = gW1[:nout]
    Wg_e = _pad_rows(gW1[nout:nout + ew], ewd)
    Wg_u = gW1[nout + ew:]
    gout = (gp["layers"][1]["W"] if two_layer else gW1).shape[1]

    grid = NN_PAD // CN

    def body(*refs):
        ns = refs[:k]
        oh_r, recv_r, send_r, u_r, e2g_r = refs[k:k + 5]
        wn = refs[k + 5:k + 5 + k]
        p = 2 * k + 5
        wng_r, wnr_r, wns_r, nb1_r = refs[p:p + 4]
        p += 4
        if two_layer:
            nw2_r, nb2_r, nlg_r, nlb_r = refs[p:p + 4]
            p += 4
        wgn_r, wge_r, wgu_r, gb1_r = refs[p:p + 4]
        p += 4
        if two_layer:
            gw2_r, gb2_r, glg_r, glb_r = refs[p:p + 4]
            p += 4
        proj_w = []
        for wrl, wsl, base in projs:
            wr_rs = refs[p:p + len(wrl)]
            p += len(wrl)
            ws_rs = refs[p:p + len(wsl)]
            p += len(wsl)
            if base is not None:
                base_rs = refs[p:p + 2]
                p += 2
            else:
                base_rs = None
            proj_w.append((wr_rs, ws_rs, base_rs))
        nnew_r, n2g_r, g_r = refs[p], refs[p + 1], refs[p + 2]
        proj_outs = refs[p + 3:]

        i = pl.program_id(0)
        h = nb1_r[...] + _dot(recv_r[...], wnr_r[...]) + _dot(send_r[...], wns_r[...])
        for nr, wr in zip(ns, wn):
            h += _dot(nr[...], wr[...])
        oh = oh_r[...]
        h += _dotp(oh, _dot(u_r[...], wng_r[...]))
        if two_layer:
            h = jnp.maximum(h, 0.0)
            h = _dot(h, nw2_r[...]) + nb2_r[...]
            h = jnp.maximum(h, 0.0)
            h = _layernorm(h, nlg_r[...], nlb_r[...])
        nnew_r[...] = h
        for t, (wr_rs, ws_rs, base_rs) in enumerate(proj_w):
            xr = base_rs[0][...] if base_rs is not None else None
            xs = base_rs[1][...] if base_rs is not None else None
            for wr_r, ws_r in zip(wr_rs, ws_rs):
                dr, ds = _dot(h, wr_r[...]), _dot(h, ws_r[...])
                xr = dr if xr is None else xr + dr
                xs = ds if xs is None else xs + ds
            proj_outs[2 * t][...] = xr
            proj_outs[2 * t + 1][...] = xs
        contrib = lax.dot_general(oh, h, (((0,), (0,)), ((), ())),
                                  preferred_element_type=F32,
                                  precision=lax.Precision.HIGHEST)

        @pl.when(i == 0)
        def _():
            n2g_r[...] = contrib

        @pl.when(i > 0)
        def _():
            n2g_r[...] += contrib

        @pl.when(i == grid - 1)
        def _():
            hg = (gb1_r[...] + _dot(n2g_r[...], wgn_r[...])
                  + _dot(e2g_r[...], wge_r[...]) + _dot(u_r[...], wgu_r[...]))
            if two_layer:
                hg = jnp.maximum(hg, 0.0)
                hg = _dot(hg, gw2_r[...]) + gb2_r[...]
                hg = jnp.maximum(hg, 0.0)
                hg = _layernorm(hg, glg_r[...], glb_r[...])
            g_r[...] = hg

    ins = [*n_parts, ohb, recv, send, u_cat, e2g, *Wn_n, Wn_g, Wn_r, Wn_s,
           nb1.reshape(1, -1)]
    if two_layer:
        l2, ln = np_["layers"][1], np_["ln"]
        ins += [l2["W"], l2["b"].reshape(1, -1),
                ln["g"].reshape(1, -1), ln["b"].reshape(1, -1)]
    ins += [Wg_n, Wg_e, Wg_u, gb1.reshape(1, -1)]
    if two_layer:
        l2, ln = gp["layers"][1], gp["ln"]
        ins += [l2["W"], l2["b"].reshape(1, -1),
                ln["g"].reshape(1, -1), ln["b"].reshape(1, -1)]

    in_specs = (
        [pl.BlockSpec((CN, p.shape[1]), lambda i: (i, 0)) for p in n_parts]
        + [pl.BlockSpec((CN, NG), lambda i: (i, 0)),
           pl.BlockSpec((CN, ewd), lambda i: (i, 0)),
           pl.BlockSpec((CN, ewd), lambda i: (i, 0))]
        + [pl.BlockSpec(a.shape, lambda i: (0, 0)) for a in ins[k + 3:]]
    )
    proj_specs, proj_out_specs, proj_out_shapes = [], [], []
    for wrl, wsl, base in projs:
        d = wrl[0].shape[1]
        for w in list(wrl) + list(wsl):
            ins.append(w)
            proj_specs.append(pl.BlockSpec(w.shape, lambda i: (0, 0)))
        if base is not None:
            for b_ in base:
                ins.append(b_)
                proj_specs.append(pl.BlockSpec((CN, d), lambda i: (i, 0)))
        proj_out_specs += [pl.BlockSpec((CN, d), lambda i: (i, 0))] * 2
        proj_out_shapes += [jax.ShapeDtypeStruct((NN_PAD, d), F32)] * 2
    in_specs = list(in_specs) + proj_specs
    out_specs = (pl.BlockSpec((CN, nout), lambda i: (i, 0)),
                 pl.BlockSpec((NG, nout), lambda i: (0, 0)),
                 pl.BlockSpec((NG, gout), lambda i: (0, 0)),
                 *proj_out_specs)
    out = pl.pallas_call(
        body,
        grid=(grid,),
        in_specs=in_specs,
        out_specs=out_specs,
        out_shape=(jax.ShapeDtypeStruct((NN_PAD, nout), F32),
                   jax.ShapeDtypeStruct((NG, nout), F32),
                   jax.ShapeDtypeStruct((NG, gout), F32),
                   *proj_out_shapes),
    )(*ins)
    proj_pairs = [(out[3 + 2 * t], out[4 + 2 * t]) for t in range(len(projs))]
    return out[0], out[2], proj_pairs


def _split_rs(eW1, widths):
    nn = sum(widths)
    Wr, Ws = eW1[:nn], eW1[nn:2 * nn]
    wr_parts, ws_parts = [], []
    o = 0
    for w in widths:
        wr_parts.append(Wr[o:o + w])
        ws_parts.append(Ws[o:o + w])
        o += w
    return wr_parts, ws_parts


def _run_block(bp, ea_parts, n_parts, u_cat, colg, rowg, ohb, ohbr, zeros,
               xr, xs, projs):
    eW1 = bp["edge"]["layers"][0]["W"]
    eb1 = bp["edge"]["layers"][0]["b"]
    nn = sum(p.shape[1] for p in n_parts)
    ne = sum(p.shape[1] for p in ea_parts)
    two_layer = bp["edge"]["ln"] is not None or len(bp["edge"]["layers"]) > 1
    ew = eW1.shape[1] if not two_layer else bp["edge"]["layers"][1]["W"].shape[1]
    We_cat = eW1[2 * nn:2 * nn + ne]
    Wg = eW1[2 * nn + ne:]
    eb1 = eb1.reshape(1, -1)
    We_parts = []
    o = 0
    for p in ea_parts:
        We_parts.append(We_cat[o:o + p.shape[1]])
        o += p.shape[1]

    gsum = _gather_pallas(xr, xs, colg, rowg)
    l2 = bp["edge"]["layers"][1] if two_layer else None
    ln = bp["edge"]["ln"]
    e_new, e2g = _edge_call(
        gsum, ea_parts, ohbr, u_cat, Wg, We_parts, eb1,
        None if l2 is None else {"W": l2["W"], "b": l2["b"].reshape(1, -1)},
        None if ln is None else {"g": ln["g"].reshape(1, -1), "b": ln["b"].reshape(1, -1)})
    recv, send = _scatter_pallas(e_new, colg, rowg, zeros[:, :e_new.shape[1]])
    n_new, g_new, proj_pairs = _node_global_call(
        n_parts, ohb, recv, send, u_cat, e2g, bp["node"], bp["global"], ew,
        projs)
    return e_new, n_new, g_new, proj_pairs


def kernel(edge_attr, x, u, params, edge_index, batch):
    row = edge_index[0].astype(jnp.int32)
    col = edge_index[1].astype(jnp.int32)
    pad = jnp.full((NE_PAD - NE,), DUMP, jnp.int32)
    rowg = jnp.concatenate([row, pad]).reshape(G_EDGE, 128)
    colg = jnp.concatenate([col, pad]).reshape(G_EDGE, 128)

    gids = jnp.arange(NG, dtype=jnp.int32)
    ohb = jnp.zeros((NN_PAD, NG), F32).at[:NN].set(
        (batch[:, None] == gids[None, :]).astype(F32))
    ohbr = jnp.zeros((NE_PAD, NG), F32).at[:NE].set(
        (batch[row][:, None] == gids[None, :]).astype(F32))
    xp = jnp.zeros((NN_PAD, x.shape[1]), F32).at[:NN].set(x)
    eap = jnp.zeros((NE_PAD, edge_attr.shape[1]), F32).at[:NE].set(edge_attr)
    zeros = jnp.zeros((NN_PAD, 128), F32)

    args = (colg, rowg, ohb, ohbr, zeros)

    wr_p, ws_p = _split_rs(params["processor"]["edge"]["layers"][0]["W"],
                           [128, 128])
    wr_d, ws_d = _split_rs(params["decoder"]["edge"]["layers"][0]["W"], [128])
    wr_o, ws_o = _split_rs(params["output"]["edge"]["layers"][0]["W"], [128])

    wr_e, ws_e = _split_rs(params["encoder"]["edge"]["layers"][0]["W"],
                           [x.shape[1]])
    xr, xs = _proj_call([xp], wr_e, ws_e)
    e0, n0, g0, pr = _run_block(
        params["encoder"], [eap], [xp], u, *args, xr, xs,
        [(wr_p[:1], ws_p[:1], None), (wr_p, ws_p, None)])
    (xr0p, xs0p), (xr_n, xs_n) = pr
    e, n, g = e0, n0, g0
    out = None
    for it in range(3):
        projs = [(wr_d, ws_d, None)]
        if it < 2:
            projs.append((wr_p[1:], ws_p[1:], (xr0p, xs0p)))
        e, n, g, pr = _run_block(params["processor"], [e0, e], [n0, n],
                                 jnp.concatenate([g0, g], axis=1), *args,
                                 xr_n, xs_n, projs)
        xr_d, xs_d = pr[0]
        if it < 2:
            xr_n, xs_n = pr[1]
        ed, nd, gd, pr = _run_block(params["decoder"], [e], [n], g, *args,
                                    xr_d, xs_d, [(wr_o, ws_o, None)])
        xr_o_, xs_o_ = pr[0]
        out = _run_block(params["output"], [ed], [nd], gd, *args,
                         xr_o_, xs_o_, [])
    return out[0][:NE, :16], out[1][:NN], out[2]


# speedup vs baseline: 1.0119x; 1.0119x over previous
import functools

import jax
import jax.numpy as jnp
from jax import lax
from jax.experimental import pallas as pl
from jax.experimental.pallas import tpu as pltpu
from jax.experimental.pallas import tpu_sc as plsc

NE, NN, NG = 160000, 10000, 16
NE_PAD = 163840
NN_PAD = 10240
DUMP = NN
NWORK = 32
G_EDGE = NE_PAD // 128
CE = 2048
CN = 2048
F32 = jnp.float32


def _gather_pallas(xr, xs, colg, rowg):
    W = xr.shape[1]
    GPT = G_EDGE // NWORK
    mesh = plsc.VectorSubcoreMesh(core_axis_name="c", subcore_axis_name="s")

    NSLOT = 2
    untiled = W < 128

    def body(xr_h, xs_h, col_h, row_h, gsum_h, col_v, row_v, *bufsem):
        bufa = bufsem[0:NSLOT]
        bufb = bufsem[NSLOT:2 * NSLOT]
        sems = bufsem[2 * NSLOT:3 * NSLOT]
        wid = lax.axis_index("s") * 2 + lax.axis_index("c")
        gbase = wid * GPT
        pltpu.sync_copy(col_h.at[pl.ds(gbase, GPT)], col_v)
        pltpu.sync_copy(row_h.at[pl.ds(gbase, GPT)], row_v)

        def fire(j, s):
            pltpu.async_copy(xr_h.at[col_v.at[j]], bufa[s], sems[s])
            pltpu.async_copy(xs_h.at[row_v.at[j]], bufb[s], sems[s])

        def drain_store(j, s):
            eb = (gbase + j) * 128
            pltpu.make_async_copy(xr_h.at[col_v.at[j]], bufa[s], sems[s]).wait()
            pltpu.make_async_copy(xs_h.at[row_v.at[j]], bufb[s], sems[s]).wait()

            def addrow(r, c):
                for q in range(W // 16):
                    sl = pl.ds(q * 16, 16)
                    bufa[s][r, sl] = bufa[s][r, sl] + bufb[s][r, sl]
                return c

            lax.fori_loop(0, 128, addrow, 0)
            pltpu.sync_copy(bufa[s], gsum_h.at[pl.ds(eb, 128)])

        for p in range(NSLOT - 1):
            fire(p, p)

        def step(j, c):
            for s in range(NSLOT):
                @pl.when(jnp.logical_and(j + NSLOT - 1 < GPT,
                                         (j + NSLOT - 1) % NSLOT == s))
                def _(s=s):
                    fire(j + NSLOT - 1, s)

            for s in range(NSLOT):
                @pl.when(j % NSLOT == s)
                def _(s=s):
                    drain_store(j, s)

            return c

        lax.fori_loop(0, GPT, step, 0)

    f = pl.kernel(
        body,
        out_type=jax.ShapeDtypeStruct((NE_PAD, W), F32),
        mesh=mesh,
        scratch_types=(
            [pltpu.VMEM((GPT, 128), jnp.int32)] * 2
            + [pltpu.VMEM((128, W), F32)] * (2 * NSLOT)
            + [pltpu.SemaphoreType.DMA] * NSLOT
        ),
        compiler_params=pltpu.CompilerParams(use_tc_tiling_on_sc=False)
        if untiled else None,
    )
    return f(xr, xs, colg, rowg)


def _scatter_pallas(e_new, colg, rowg, zeros):
    W = e_new.shape[1]
    RPT = NN_PAD // 16
    GPT = G_EDGE // 16
    untiled = W < 128
    mesh = plsc.VectorSubcoreMesh(core_axis_name="c", subcore_axis_name="s")

    def body(e_h, col_h, row_h, z_h, recv_h, send_h, acc, idx_v,
             ebuf0, ebuf1, seml0, seml1):
        cid = lax.axis_index("c")
        sid = lax.axis_index("s")
        pltpu.sync_copy(z_h.at[pl.ds(sid * RPT, RPT)], acc.at[pl.ds(sid * RPT, RPT)])

        @pl.when(cid == 0)
        def _():
            pltpu.sync_copy(col_h.at[pl.ds(sid * GPT, GPT)], idx_v)

        @pl.when(cid == 1)
        def _():
            pltpu.sync_copy(row_h.at[pl.ds(sid * GPT, GPT)], idx_v)

        plsc.subcore_barrier()

        def load(j, buf, sem):
            eb = (sid * GPT + j) * 128
            pltpu.async_copy(e_h.at[pl.ds(eb, 128)], buf, sem)

        def drain_scatter(j, buf, sem):
            eb = (sid * GPT + j) * 128
            pltpu.make_async_copy(e_h.at[pl.ds(eb, 128)], buf, sem).wait()
            pltpu.sync_copy(buf, acc.at[idx_v.at[j]], add=True)

        load(0, ebuf0, seml0)

        def step(j, c):
            @pl.when(jnp.logical_and(j + 1 < GPT, j % 2 == 0))
            def _():
                load(j + 1, ebuf1, seml1)

            @pl.when(jnp.logical_and(j + 1 < GPT, j % 2 == 1))
            def _():
                load(j + 1, ebuf0, seml0)

            @pl.when(j % 2 == 0)
            def _():
                drain_scatter(j, ebuf0, seml0)

            @pl.when(j % 2 == 1)
            def _():
                drain_scatter(j, ebuf1, seml1)

            return c

        lax.fori_loop(0, GPT, step, 0)
        plsc.subcore_barrier()

        @pl.when(cid == 0)
        def _():
            pltpu.sync_copy(acc.at[pl.ds(sid * RPT, RPT)], recv_h.at[pl.ds(sid * RPT, RPT)])

        @pl.when(cid == 1)
        def _():
            pltpu.sync_copy(acc.at[pl.ds(sid * RPT, RPT)], send_h.at[pl.ds(sid * RPT, RPT)])

    f = pl.kernel(
        body,
        out_type=(jax.ShapeDtypeStruct((NN_PAD, W), F32),
                  jax.ShapeDtypeStruct((NN_PAD, W), F32)),
        mesh=mesh,
        scratch_types=[
            pltpu.VMEM_SHARED((NN_PAD, W), F32),
            pltpu.VMEM((GPT, 128), jnp.int32),
            pltpu.VMEM((128, W), F32),
            pltpu.VMEM((128, W), F32),
            pltpu.SemaphoreType.DMA,
            pltpu.SemaphoreType.DMA,
        ],
        compiler_params=pltpu.CompilerParams(use_tc_tiling_on_sc=False)
        if untiled else None,
    )
    return f(e_new, colg, rowg, zeros)


def _layernorm(h, g, b):
    m = jnp.mean(h, axis=-1, keepdims=True)
    v = jnp.mean((h - m) ** 2, axis=-1, keepdims=True)
    return (h - m) * lax.rsqrt(v + 1e-5) * g + b


def _dot(a, b):
    return jnp.dot(a, b, preferred_element_type=F32)


def _dotp(a, b):
    return jnp.dot(a, b, preferred_element_type=F32,
                   precision=lax.Precision.HIGHEST)


def _proj_call(n_parts, Wr_parts, Ws_parts):
    lat = Wr_parts[0].shape[1]
    k = len(n_parts)
    grid = NN_PAD // CN

    def body(*refs):
        ns = refs[:k]
        wr = refs[k:2 * k]
        ws = refs[2 * k:3 * k]
        xr_r, xs_r = refs[3 * k], refs[3 * k + 1]
        xr = _dot(ns[0][...], wr[0][...])
        xs = _dot(ns[0][...], ws[0][...])
        for i in range(1, k):
            xr += _dot(ns[i][...], wr[i][...])
            xs += _dot(ns[i][...], ws[i][...])
        xr_r[...] = xr
        xs_r[...] = xs

    in_specs = (
        [pl.BlockSpec((CN, p.shape[1]), lambda i: (i, 0)) for p in n_parts]
        + [pl.BlockSpec(w.shape, lambda i: (0, 0)) for w in Wr_parts]
        + [pl.BlockSpec(w.shape, lambda i: (0, 0)) for w in Ws_parts]
    )
    out_specs = (pl.BlockSpec((CN, lat), lambda i: (i, 0)),) * 2
    return pl.pallas_call(
        body,
        grid=(grid,),
        in_specs=in_specs,
        out_specs=out_specs,
        out_shape=(jax.ShapeDtypeStruct((NN_PAD, lat), F32),) * 2,
    )(*n_parts, *Wr_parts, *Ws_parts)


def _edge_call(gsum, ea_parts, ohbr, u_cat, Wg, We_parts, b1, l2, ln):
    k = len(ea_parts)
    two_layer = l2 is not None
    dout = l2["W"].shape[1] if two_layer else b1.shape[1]
    grid = NE_PAD // CE

    def body(*refs):
        gsum_r = refs[0]
        eas = refs[1:1 + k]
        oh_r, u_r, wg_r = refs[1 + k], refs[2 + k], refs[3 + k]
        wes = refs[4 + k:4 + 2 * k]
        b1_r = refs[4 + 2 * k]
        pos = 5 + 2 * k
        if two_layer:
            w2_r, b2_r, g_r, be_r = refs[pos:pos + 4]
            pos += 4
        enew_r, e2g_r = refs[pos], refs[pos + 1]

        i = pl.program_id(0)
        h = gsum_r[...] + b1_r[...]
        for ear, wer in zip(eas, wes):
            h += _dot(ear[...], wer[...])
        ug = _dot(u_r[...], wg_r[...])
        oh = oh_r[...]
        h += _dotp(oh, ug)
        if two_layer:
            h = jnp.maximum(h, 0.0)
            h = _dot(h, w2_r[...]) + b2_r[...]
            h = jnp.maximum(h, 0.0)
            h = _layernorm(h, g_r[...], be_r[...])
        enew_r[...] = h
        contrib = lax.dot_general(oh, h, (((0,), (0,)), ((), ())),
                                  preferred_element_type=F32,
                                  precision=lax.Precision.HIGHEST)

        @pl.when(i == 0)
        def _():
            e2g_r[...] = contrib

        @pl.when(i > 0)
        def _():
            e2g_r[...] += contrib

    ins = [gsum, *ea_parts, ohbr, u_cat, Wg, *We_parts, b1]
    if two_layer:
        ins += [l2["W"], l2["b"], ln["g"], ln["b"]]
    in_specs = (
        [pl.BlockSpec((CE, gsum.shape[1]), lambda i: (i, 0))]
        + [pl.BlockSpec((CE, p.shape[1]), lambda i: (i, 0)) for p in ea_parts]
        + [pl.BlockSpec((CE, NG), lambda i: (i, 0))]
        + [pl.BlockSpec(a.shape, lambda i: (0, 0))
           for a in ins[2 + k:]]
    )
    out_specs = (pl.BlockSpec((CE, dout), lambda i: (i, 0)),
                 pl.BlockSpec((NG, dout), lambda i: (0, 0)))
    return pl.pallas_call(
        body,
        grid=(grid,),
        in_specs=in_specs,
        out_specs=out_specs,
        out_shape=(jax.ShapeDtypeStruct((NE_PAD, dout), F32),
                   jax.ShapeDtypeStruct((NG, dout), F32)),
    )(*ins)


def _pad_cols(w, to):
    return w if w.shape[1] == to else jnp.pad(w, ((0, 0), (0, to - w.shape[1])))


def _pad_rows(w, to):
    return w if w.shape[0] == to else jnp.pad(w, ((0, to - w.shape[0]), (0, 0)))


def _node_global_call(n_parts, ohb, recv, send, u_cat, e2g, np_, gp, ew):
    k = len(n_parts)
    nW1, nb1 = np_["layers"][0]["W"], np_["layers"][0]["b"]
    two_layer = np_["ln"] is not None or len(np_["layers"]) > 1
    nn = sum(p.shape[1] for p in n_parts)
    ng = u_cat.shape[1]
    ewd = recv.shape[1]
    Wn_n = [nW1[sum(p.shape[1] for p in n_parts[:i]):
                sum(p.shape[1] for p in n_parts[:i + 1])] for i in range(k)]
    Wn_g = nW1[nn:nn + ng]
    Wn_r = _pad_rows(nW1[nn + ng:nn + ng + ew], ewd)
    Wn_s = _pad_rows(nW1[nn + ng + ew:], ewd)
    nout = (np_["layers"][1]["W"] if two_layer else nW1).shape[1]

    gW1, gb1 = gp["layers"][0]["W"], gp["layers"][0]["b"]
    Wg_n = gW1[:nout]
    Wg_e = _pad_rows(gW1[nout:nout + ew], ewd)
    Wg_u = gW1[nout + ew:]
    gout = (gp["layers"][1]["W"] if two_layer else gW1).shape[1]

    grid = NN_PAD // CN

    def body(*refs):
        ns = refs[:k]
        oh_r, recv_r, send_r, u_r, e2g_r = refs[k:k + 5]
        wn = refs[k + 5:k + 5 + k]
        p = 2 * k + 5
        wng_r, wnr_r, wns_r, nb1_r = refs[p:p + 4]
        p += 4
        if two_layer:
            nw2_r, nb2_r, nlg_r, nlb_r = refs[p:p + 4]
            p += 4
        wgn_r, wge_r, wgu_r, gb1_r = refs[p:p + 4]
        p += 4
        if two_layer:
            gw2_r, gb2_r, glg_r, glb_r = refs[p:p + 4]
            p += 4
        nnew_r, n2g_r, g_r = refs[p], refs[p + 1], refs[p + 2]

        i = pl.program_id(0)
        h = nb1_r[...] + _dot(recv_r[...], wnr_r[...]) + _dot(send_r[...], wns_r[...])
        for nr, wr in zip(ns, wn):
            h += _dot(nr[...], wr[...])
        oh = oh_r[...]
        h += _dotp(oh, _dot(u_r[...], wng_r[...]))
        if two_layer:
            h = jnp.maximum(h, 0.0)
            h = _dot(h, nw2_r[...]) + nb2_r[...]
            h = jnp.maximum(h, 0.0)
            h = _layernorm(h, nlg_r[...], nlb_r[...])
        nnew_r[...] = h
        contrib = lax.dot_general(oh, h, (((0,), (0,)), ((), ())),
                                  preferred_element_type=F32,
                                  precision=lax.Precision.HIGHEST)

        @pl.when(i == 0)
        def _():
            n2g_r[...] = contrib

        @pl.when(i > 0)
        def _():
            n2g_r[...] += contrib

        @pl.when(i == grid - 1)
        def _():
            hg = (gb1_r[...] + _dot(n2g_r[...], wgn_r[...])
                  + _dot(e2g_r[...], wge_r[...]) + _dot(u_r[...], wgu_r[...]))
            if two_layer:
                hg = jnp.maximum(hg, 0.0)
                hg = _dot(hg, gw2_r[...]) + gb2_r[...]
                hg = jnp.maximum(hg, 0.0)
                hg = _layernorm(hg, glg_r[...], glb_r[...])
            g_r[...] = hg

    ins = [*n_parts, ohb, recv, send, u_cat, e2g, *Wn_n, Wn_g, Wn_r, Wn_s,
           nb1.reshape(1, -1)]
    if two_layer:
        l2, ln = np_["layers"][1], np_["ln"]
        ins += [l2["W"], l2["b"].reshape(1, -1),
                ln["g"].reshape(1, -1), ln["b"].reshape(1, -1)]
    ins += [Wg_n, Wg_e, Wg_u, gb1.reshape(1, -1)]
    if two_layer:
        l2, ln = gp["layers"][1], gp["ln"]
        ins += [l2["W"], l2["b"].reshape(1, -1),
                ln["g"].reshape(1, -1), ln["b"].reshape(1, -1)]

    in_specs = (
        [pl.BlockSpec((CN, p.shape[1]), lambda i: (i, 0)) for p in n_parts]
        + [pl.BlockSpec((CN, NG), lambda i: (i, 0)),
           pl.BlockSpec((CN, ewd), lambda i: (i, 0)),
           pl.BlockSpec((CN, ewd), lambda i: (i, 0))]
        + [pl.BlockSpec(a.shape, lambda i: (0, 0)) for a in ins[k + 3:]]
    )
    out_specs = (pl.BlockSpec((CN, nout), lambda i: (i, 0)),
                 pl.BlockSpec((NG, nout), lambda i: (0, 0)),
                 pl.BlockSpec((NG, gout), lambda i: (0, 0)))
    out = pl.pallas_call(
        body,
        grid=(grid,),
        in_specs=in_specs,
        out_specs=out_specs,
        out_shape=(jax.ShapeDtypeStruct((NN_PAD, nout), F32),
                   jax.ShapeDtypeStruct((NG, nout), F32),
                   jax.ShapeDtypeStruct((NG, gout), F32)),
    )(*ins)
    return out[0], out[2]


def _run_block(bp, ea_parts, n_parts, u_cat, colg, rowg, ohb, ohbr, zeros):
    eW1 = bp["edge"]["layers"][0]["W"]
    eb1 = bp["edge"]["layers"][0]["b"]
    nn = sum(p.shape[1] for p in n_parts)
    ne = sum(p.shape[1] for p in ea_parts)
    two_layer = bp["edge"]["ln"] is not None or len(bp["edge"]["layers"]) > 1
    ew = eW1.shape[1] if not two_layer else bp["edge"]["layers"][1]["W"].shape[1]
    latp = eW1.shape[1]
    Wr, Ws = eW1[:nn], eW1[nn:2 * nn]
    We_cat = eW1[2 * nn:2 * nn + ne]
    Wg = _pad_cols(eW1[2 * nn + ne:], latp)
    eb1 = _pad_cols(eb1.reshape(1, -1), latp)
    Wr_parts, Ws_parts, We_parts = [], [], []
    o = 0
    for p in n_parts:
        Wr_parts.append(_pad_cols(Wr[o:o + p.shape[1]], latp))
        Ws_parts.append(_pad_cols(Ws[o:o + p.shape[1]], latp))
        o += p.shape[1]
    o = 0
    for p in ea_parts:
        We_parts.append(_pad_cols(We_cat[o:o + p.shape[1]], latp))
        o += p.shape[1]

    xr, xs = _proj_call(n_parts, Wr_parts, Ws_parts)
    gsum = _gather_pallas(xr, xs, colg, rowg)
    l2 = bp["edge"]["layers"][1] if two_layer else None
    ln = bp["edge"]["ln"]
    e_new, e2g = _edge_call(
        gsum, ea_parts, ohbr, u_cat, Wg, We_parts, eb1,
        None if l2 is None else {"W": l2["W"], "b": l2["b"].reshape(1, -1)},
        None if ln is None else {"g": ln["g"].reshape(1, -1), "b": ln["b"].reshape(1, -1)})
    recv, send = _scatter_pallas(e_new, colg, rowg, zeros[:, :e_new.shape[1]])
    n_new, g_new = _node_global_call(n_parts, ohb, recv, send, u_cat, e2g,
                                     bp["node"], bp["global"], ew)
    return e_new, n_new, g_new


def kernel(edge_attr, x, u, params, edge_index, batch):
    row = edge_index[0].astype(jnp.int32)
    col = edge_index[1].astype(jnp.int32)
    pad = jnp.full((NE_PAD - NE,), DUMP, jnp.int32)
    rowg = jnp.concatenate([row, pad]).reshape(G_EDGE, 128)
    colg = jnp.concatenate([col, pad]).reshape(G_EDGE, 128)

    gids = jnp.arange(NG, dtype=jnp.int32)
    ohb = jnp.zeros((NN_PAD, NG), F32).at[:NN].set(
        (batch[:, None] == gids[None, :]).astype(F32))
    ohbr = jnp.zeros((NE_PAD, NG), F32).at[:NE].set(
        (batch[row][:, None] == gids[None, :]).astype(F32))
    xp = jnp.zeros((NN_PAD, x.shape[1]), F32).at[:NN].set(x)
    eap = jnp.zeros((NE_PAD, edge_attr.shape[1]), F32).at[:NE].set(edge_attr)
    zeros = jnp.zeros((NN_PAD, 128), F32)

    args = (colg, rowg, ohb, ohbr, zeros)
    e0, n0, g0 = _run_block(params["encoder"], [eap], [xp], u, *args)
    e, n, g = e0, n0, g0
    out = None
    for _ in range(3):
        e, n, g = _run_block(params["processor"], [e0, e], [n0, n],
                             jnp.concatenate([g0, g], axis=1), *args)
        ed, nd, gd = _run_block(params["decoder"], [e], [n], g, *args)
        out = _run_block(params["output"], [ed], [nd], gd, *args)
    return out[0][:NE, :16], out[1][:NN], out[2]
